# Initial kernel scaffold; baseline (speedup 1.0000x reference)
import jax
import jax.numpy as jnp
from jax.experimental import pallas as pl


def kernel(x, duration_predictor_output, max_len):
    raise NotImplementedError("write your pallas kernel here")


# speedup vs baseline: 3.7642x; 3.7642x over previous
import functools

import jax
import jax.numpy as jnp
from jax import lax
from jax.experimental import pallas as pl
from jax.experimental.pallas import tpu as pltpu
from jax.experimental.pallas import tpu_sc as plsc

B, T, D, P = 16, 512, 256, 2048
NROWS = B * P
VROWS = B * T
PAD = 8
TBL = VROWS + PAD

NW = 32
ROWS_PER_W = NROWS // NW
CH = 128
NCH = ROWS_PER_W // CH


def _idx_body(dur_ref, idx_ref):
    b = pl.program_id(0)
    dur_col = dur_ref[0].astype(jnp.float32)
    ii = lax.broadcasted_iota(jnp.int32, (T, T), 0)
    jj = lax.broadcasted_iota(jnp.int32, (T, T), 1)
    ltri = (jj <= ii).astype(jnp.float32)
    cum_col = jnp.dot(ltri, dur_col,
                      preferred_element_type=jnp.float32)
    pos_row = lax.broadcasted_iota(jnp.int32, (1, P), 1).astype(
        jnp.float32)
    cmp = (cum_col <= pos_row).astype(jnp.int32)
    idx = jnp.sum(cmp, axis=0, keepdims=True)
    total = lax.slice(cum_col, (T - 1, 0), (T, 1))
    valid = pos_row < total
    flat = jnp.where(valid,
                     b * T + jnp.minimum(idx, T - 1),
                     VROWS).astype(jnp.int32)
    idx_ref[0] = flat


def _build_idx(duration):
    dur_col = duration.reshape(B, T, 1)
    out = pl.pallas_call(
        _idx_body,
        grid=(B,),
        in_specs=[pl.BlockSpec((1, T, 1), lambda i: (i, 0, 0))],
        out_specs=pl.BlockSpec((1, 1, P), lambda i: (i, 0, 0)),
        out_shape=jax.ShapeDtypeStruct((B, 1, P), jnp.int32),
    )(dur_col)
    return out.reshape(NW, NCH, CH)


_XBLK = 8
_NXBLK = TBL // _XBLK


def _xpad_body(x_ref, out_ref):
    i = pl.program_id(0)
    out_ref[...] = jnp.where(i < VROWS // _XBLK, x_ref[...], 0.0)


def _build_xpad(x_flat):
    return pl.pallas_call(
        _xpad_body,
        grid=(_NXBLK,),
        in_specs=[pl.BlockSpec(
            (_XBLK, D), lambda i: (jnp.minimum(i, VROWS // _XBLK - 1), 0))],
        out_specs=pl.BlockSpec((_XBLK, D), lambda i: (i, 0)),
        out_shape=jax.ShapeDtypeStruct((TBL, D), jnp.float32),
    )(x_flat)


@functools.lru_cache(maxsize=None)
def _make_sc_gather():
    mesh = plsc.VectorSubcoreMesh(
        core_axis_name="c", subcore_axis_name="s",
        num_cores=2, num_subcores=16)

    @functools.partial(
        pl.kernel,
        out_type=jax.ShapeDtypeStruct((NROWS, D), jnp.float32),
        mesh=mesh,
        scratch_types=[
            pltpu.VMEM((NCH, CH), jnp.int32),
            pltpu.VMEM((CH, D), jnp.float32),
            pltpu.SemaphoreType.DMA,
        ],
    )
    def _sc_gather(xpad_hbm, idx_hbm, out_hbm, idx_v, rows_v, sem):
        wid = lax.axis_index("s") * 2 + lax.axis_index("c")
        pltpu.sync_copy(idx_hbm.at[wid], idx_v)
        base = wid * ROWS_PER_W
        for j in range(NCH):
            pltpu.async_copy(xpad_hbm.at[idx_v.at[j]], rows_v, sem).wait()
            pltpu.sync_copy(rows_v, out_hbm.at[pl.ds(base + j * CH, CH)])

    return _sc_gather


def kernel(x, duration_predictor_output, max_len):
    x_flat = x.reshape(VROWS, D)
    xpad = _build_xpad(x_flat)
    idx3 = _build_idx(duration_predictor_output)
    out = _make_sc_gather()(xpad, idx3)
    return out.reshape(B, P, D)


# speedup vs baseline: 6.0957x; 1.6194x over previous
import functools

import jax
import jax.numpy as jnp
from jax import lax
from jax.experimental import pallas as pl
from jax.experimental.pallas import tpu as pltpu
from jax.experimental.pallas import tpu_sc as plsc

B, T, D, P = 16, 512, 256, 2048
NROWS = B * P
VROWS = B * T
PAD = 1024
TBL = VROWS + PAD

NW = 32
ROWS_PER_W = NROWS // NW
CH = 128
NCH = ROWS_PER_W // CH


def _idx_body(dur_ref, idx_ref):
    b = pl.program_id(0)
    dur_col = dur_ref[0].astype(jnp.float32)
    ii = lax.broadcasted_iota(jnp.int32, (T, T), 0)
    jj = lax.broadcasted_iota(jnp.int32, (T, T), 1)
    ltri = (jj <= ii).astype(jnp.float32)
    cum_col = jnp.dot(ltri, dur_col,
                      preferred_element_type=jnp.float32)
    pos_row = lax.broadcasted_iota(jnp.int32, (1, P), 1).astype(
        jnp.float32)
    cmp = (cum_col <= pos_row).astype(jnp.int32)
    idx = jnp.sum(cmp, axis=0, keepdims=True)
    total = lax.slice(cum_col, (T - 1, 0), (T, 1))
    valid = pos_row < total
    flat = jnp.where(valid,
                     b * T + jnp.minimum(idx, T - 1),
                     VROWS).astype(jnp.int32)
    idx_ref[0] = flat


def _build_idx(duration):
    dur_col = duration.reshape(B, T, 1)
    out = pl.pallas_call(
        _idx_body,
        grid=(B,),
        in_specs=[pl.BlockSpec((1, T, 1), lambda i: (i, 0, 0))],
        out_specs=pl.BlockSpec((1, 1, P), lambda i: (i, 0, 0)),
        out_shape=jax.ShapeDtypeStruct((B, 1, P), jnp.int32),
    )(dur_col)
    return out.reshape(NW, NCH, CH)


_XBLK = 1024
_NXBLK = TBL // _XBLK


def _xpad_body(x_ref, out_ref):
    i = pl.program_id(0)
    out_ref[...] = jnp.where(i < VROWS // _XBLK, x_ref[...], 0.0)


def _build_xpad(x_flat):
    return pl.pallas_call(
        _xpad_body,
        grid=(_NXBLK,),
        in_specs=[pl.BlockSpec(
            (_XBLK, D), lambda i: (jnp.minimum(i, VROWS // _XBLK - 1), 0))],
        out_specs=pl.BlockSpec((_XBLK, D), lambda i: (i, 0)),
        out_shape=jax.ShapeDtypeStruct((TBL, D), jnp.float32),
    )(x_flat)


NBUF = 3


@functools.lru_cache(maxsize=None)
def _make_sc_gather():
    mesh = plsc.VectorSubcoreMesh(
        core_axis_name="c", subcore_axis_name="s",
        num_cores=2, num_subcores=16)

    @functools.partial(
        pl.kernel,
        out_type=jax.ShapeDtypeStruct((NROWS, D), jnp.float32),
        mesh=mesh,
        scratch_types=[
            pltpu.VMEM((NCH, CH), jnp.int32),
            pltpu.VMEM((NBUF, CH, D), jnp.float32),
            pltpu.SemaphoreType.DMA((NBUF,)),
            pltpu.SemaphoreType.DMA((NBUF,)),
        ],
    )
    def _sc_gather(xpad_hbm, idx_hbm, out_hbm, idx_v, rows_v, gsem, wsem):
        wid = lax.axis_index("s") * 2 + lax.axis_index("c")
        pltpu.sync_copy(idx_hbm.at[wid], idx_v)
        base = wid * ROWS_PER_W

        def gather(j):
            return pltpu.async_copy(
                xpad_hbm.at[idx_v.at[j]], rows_v.at[j % NBUF],
                gsem.at[j % NBUF])

        def put(j):
            return pltpu.async_copy(
                rows_v.at[j % NBUF],
                out_hbm.at[pl.ds(base + j * CH, CH)],
                wsem.at[j % NBUF])

        LOOK = 2
        g = [None] * NCH
        w = [None] * NCH
        for j in range(min(LOOK, NCH)):
            g[j] = gather(j)
        for j in range(NCH):
            g[j].wait()
            w[j] = put(j)
            nxt = j + LOOK
            if nxt < NCH:
                if nxt - NBUF >= 0:
                    w[nxt - NBUF].wait()
                g[nxt] = gather(nxt)
        for j in range(max(0, NCH - NBUF), NCH):
            w[j].wait()

    return _sc_gather


def kernel(x, duration_predictor_output, max_len):
    x_flat = x.reshape(VROWS, D)
    xpad = _build_xpad(x_flat)
    idx3 = _build_idx(duration_predictor_output)
    out = _make_sc_gather()(xpad, idx3)
    return out.reshape(B, P, D)


# speedup vs baseline: 55.9378x; 9.1766x over previous
import functools

import jax
import jax.numpy as jnp
from jax import lax
from jax.experimental import pallas as pl
from jax.experimental.pallas import tpu as pltpu
from jax.experimental.pallas import tpu_sc as plsc

B, T, D, P = 16, 512, 256, 2048
NROWS = B * P
VROWS = B * T
PAD = 1024
TBL = VROWS + PAD

NW = 32
ROWS_PER_W = NROWS // NW
CH = 128
NCH = ROWS_PER_W // CH


def _idx_body(dur_ref, idx_ref):
    b = pl.program_id(0)
    dur_col = dur_ref[0].astype(jnp.float32)
    ii = lax.broadcasted_iota(jnp.int32, (T, T), 0)
    jj = lax.broadcasted_iota(jnp.int32, (T, T), 1)
    ltri = (jj <= ii).astype(jnp.float32)
    cum_col = jnp.dot(ltri, dur_col,
                      preferred_element_type=jnp.float32)
    m = lax.broadcasted_iota(jnp.int32, (1, P), 1)
    pos_i = (m // 1024 + 2 * ((m % 1024) // CH)) * CH + m % CH
    pos_row = pos_i.astype(jnp.float32)
    cmp = (cum_col <= pos_row).astype(jnp.int32)
    idx = jnp.sum(cmp, axis=0, keepdims=True)
    total = lax.slice(cum_col, (T - 1, 0), (T, 1))
    valid = pos_row < total
    flat = jnp.where(valid,
                     b * T + jnp.minimum(idx, T - 1),
                     VROWS + (pos_i % PAD)).astype(jnp.int32)
    idx_ref[0] = flat


def _build_idx(duration):
    dur_col = duration.reshape(B, T, 1)
    out = pl.pallas_call(
        _idx_body,
        grid=(B,),
        in_specs=[pl.BlockSpec((1, T, 1), lambda i: (i, 0, 0))],
        out_specs=pl.BlockSpec((1, 1, P), lambda i: (i, 0, 0)),
        out_shape=jax.ShapeDtypeStruct((B, 1, P), jnp.int32),
    )(dur_col)
    return out.reshape(B, 2, NCH, CH)


_XBLK = 1024
_NXBLK = TBL // _XBLK


def _xpad_body(x_ref, out_ref):
    i = pl.program_id(0)
    out_ref[...] = jnp.where(i < VROWS // _XBLK, x_ref[...], 0.0)


def _build_xpad(x_flat):
    return pl.pallas_call(
        _xpad_body,
        grid=(_NXBLK,),
        in_specs=[pl.BlockSpec(
            (_XBLK, D), lambda i: (jnp.minimum(i, VROWS // _XBLK - 1), 0))],
        out_specs=pl.BlockSpec((_XBLK, D), lambda i: (i, 0)),
        out_shape=jax.ShapeDtypeStruct((TBL, D), jnp.float32),
    )(x_flat)


NBUF = 3


@functools.lru_cache(maxsize=None)
def _make_sc_gather():
    mesh = plsc.VectorSubcoreMesh(
        core_axis_name="c", subcore_axis_name="s",
        num_cores=2, num_subcores=16)

    @functools.partial(
        pl.kernel,
        out_type=jax.ShapeDtypeStruct((NROWS, D), jnp.float32),
        mesh=mesh,
        scratch_types=[
            pltpu.VMEM((NCH, CH), jnp.int32),
            pltpu.VMEM((NBUF, CH, D), jnp.float32),
            pltpu.SemaphoreType.DMA((NBUF,)),
            pltpu.SemaphoreType.DMA((NBUF,)),
        ],
    )
    def _sc_gather(xpad_hbm, idx_hbm, out_hbm, idx_v, rows_v, gsem, wsem):
        wid = lax.axis_index("s") * 2 + lax.axis_index("c")
        b = wid % B
        h = wid // B
        pltpu.sync_copy(idx_hbm.at[b, h], idx_v)

        def gather(j):
            return pltpu.async_copy(
                xpad_hbm.at[idx_v.at[j]], rows_v.at[j % NBUF],
                gsem.at[j % NBUF])

        def put(j):
            return pltpu.async_copy(
                rows_v.at[j % NBUF],
                out_hbm.at[pl.ds(b * P + (h + 2 * j) * CH, CH)],
                wsem.at[j % NBUF])

        LOOK = 2
        g = [None] * NCH
        w = [None] * NCH
        for j in range(min(LOOK, NCH)):
            g[j] = gather(j)
        for j in range(NCH):
            g[j].wait()
            w[j] = put(j)
            nxt = j + LOOK
            if nxt < NCH:
                if nxt - NBUF >= 0:
                    w[nxt - NBUF].wait()
                g[nxt] = gather(nxt)
        for j in range(max(0, NCH - NBUF), NCH):
            w[j].wait()

    return _sc_gather


def kernel(x, duration_predictor_output, max_len):
    x_flat = x.reshape(VROWS, D)
    xpad = _build_xpad(x_flat)
    idx3 = _build_idx(duration_predictor_output)
    out = _make_sc_gather()(xpad, idx3)
    return out.reshape(B, P, D)


# speedup vs baseline: 63.0368x; 1.1269x over previous
import functools

import jax
import jax.numpy as jnp
from jax import lax
from jax.experimental import pallas as pl
from jax.experimental.pallas import tpu as pltpu
from jax.experimental.pallas import tpu_sc as plsc

B, T, D, P = 16, 512, 256, 2048
NROWS = B * P
VROWS = B * T
PAD = 1024
TBL = VROWS + PAD

NW = 32
ROWS_PER_W = NROWS // NW
CH = 128
NCH = ROWS_PER_W // CH


def _idx_body(ltri_ref, durT_ref, idx_ref):
    durT = durT_ref[...].astype(jnp.float32)
    cum = jnp.dot(ltri_ref[...], durT,
                  preferred_element_type=jnp.float32)
    m = lax.broadcasted_iota(jnp.int32, (1, P), 1)
    pos_i = (m // 1024 + 2 * ((m % 1024) // CH)) * CH + m % CH
    pos_row = pos_i.astype(jnp.float32)
    zrow = VROWS + (pos_i % PAD)
    for b in range(B):
        cum_b = lax.slice(cum, (0, b), (T, b + 1))
        cmp = (cum_b <= pos_row).astype(jnp.int32)
        idx = jnp.sum(cmp, axis=0, keepdims=True)
        total = lax.slice(cum_b, (T - 1, 0), (T, 1))
        valid = pos_row < total
        flat = jnp.where(valid,
                         b * T + jnp.minimum(idx, T - 1),
                         zrow).astype(jnp.int32)
        idx_ref[b] = flat


def _build_idx(duration):
    durT = duration.T
    ltri = jnp.tri(T, dtype=jnp.float32)
    out = pl.pallas_call(
        _idx_body,
        out_shape=jax.ShapeDtypeStruct((B, 1, P), jnp.int32),
    )(ltri, durT)
    return out.reshape(B, 2, NCH, CH)


_XBLK = 1024
_NXBLK = TBL // _XBLK


def _xpad_body(x_ref, out_ref):
    i = pl.program_id(0)
    out_ref[...] = jnp.where(i < VROWS // _XBLK, x_ref[...], 0.0)


def _build_xpad(x_flat):
    return pl.pallas_call(
        _xpad_body,
        grid=(_NXBLK,),
        in_specs=[pl.BlockSpec(
            (_XBLK, D), lambda i: (jnp.minimum(i, VROWS // _XBLK - 1), 0))],
        out_specs=pl.BlockSpec((_XBLK, D), lambda i: (i, 0)),
        out_shape=jax.ShapeDtypeStruct((TBL, D), jnp.float32),
    )(x_flat)


NBUF = 3


@functools.lru_cache(maxsize=None)
def _make_sc_gather():
    mesh = plsc.VectorSubcoreMesh(
        core_axis_name="c", subcore_axis_name="s",
        num_cores=2, num_subcores=16)

    @functools.partial(
        pl.kernel,
        out_type=jax.ShapeDtypeStruct((NROWS, D), jnp.float32),
        mesh=mesh,
        scratch_types=[
            pltpu.VMEM((NCH, CH), jnp.int32),
            pltpu.VMEM((NBUF, CH, D), jnp.float32),
            pltpu.SemaphoreType.DMA((NBUF,)),
            pltpu.SemaphoreType.DMA((NBUF,)),
        ],
    )
    def _sc_gather(xpad_hbm, idx_hbm, out_hbm, idx_v, rows_v, gsem, wsem):
        wid = lax.axis_index("s") * 2 + lax.axis_index("c")
        b = wid % B
        h = wid // B
        pltpu.sync_copy(idx_hbm.at[b, h], idx_v)

        def gather(j):
            return pltpu.async_copy(
                xpad_hbm.at[idx_v.at[j]], rows_v.at[j % NBUF],
                gsem.at[j % NBUF])

        def put(j):
            return pltpu.async_copy(
                rows_v.at[j % NBUF],
                out_hbm.at[pl.ds(b * P + (h + 2 * j) * CH, CH)],
                wsem.at[j % NBUF])

        LOOK = 2
        g = [None] * NCH
        w = [None] * NCH
        for j in range(min(LOOK, NCH)):
            g[j] = gather(j)
        for j in range(NCH):
            g[j].wait()
            w[j] = put(j)
            nxt = j + LOOK
            if nxt < NCH:
                if nxt - NBUF >= 0:
                    w[nxt - NBUF].wait()
                g[nxt] = gather(nxt)
        for j in range(max(0, NCH - NBUF), NCH):
            w[j].wait()

    return _sc_gather


def kernel(x, duration_predictor_output, max_len):
    x_flat = x.reshape(VROWS, D)
    xpad = _build_xpad(x_flat)
    idx3 = _build_idx(duration_predictor_output)
    out = _make_sc_gather()(xpad, idx3)
    return out.reshape(B, P, D)


# speedup vs baseline: 65.0761x; 1.0324x over previous
import functools

import jax
import jax.numpy as jnp
from jax import lax
from jax.experimental import pallas as pl
from jax.experimental.pallas import tpu as pltpu
from jax.experimental.pallas import tpu_sc as plsc

B, T, D, P = 16, 512, 256, 2048
NROWS = B * P
VROWS = B * T
PAD = 1024
TBL = VROWS + PAD

NW = 32
ROWS_PER_W = NROWS // NW
CH = 128
NCH = ROWS_PER_W // CH


def _idx_body(ltri_ref, durT_ref, idx_ref, tot_ref):
    durT = durT_ref[...].astype(jnp.float32)
    cum = jnp.dot(ltri_ref[...], durT,
                  preferred_element_type=jnp.float32)
    m = lax.broadcasted_iota(jnp.int32, (1, P), 1)
    pos_i = (m // 1024 + 2 * ((m % 1024) // CH)) * CH + m % CH
    pos_row = pos_i.astype(jnp.float32)
    zrow = VROWS + (pos_i % PAD)
    for b in range(B):
        cum_b = lax.slice(cum, (0, b), (T, b + 1))
        cmp = (cum_b <= pos_row).astype(jnp.int32)
        idx = jnp.sum(cmp, axis=0, keepdims=True)
        total = lax.slice(cum_b, (T - 1, 0), (T, 1))
        valid = pos_row < total
        flat = jnp.where(valid,
                         b * T + jnp.minimum(idx, T - 1),
                         zrow).astype(jnp.int32)
        idx_ref[b] = flat
        tot_ref[b] = jnp.broadcast_to(total.astype(jnp.int32), (1, 16))


def _build_idx(duration):
    durT = duration.T
    ltri = jnp.tri(T, dtype=jnp.float32)
    out, tot = pl.pallas_call(
        _idx_body,
        out_shape=[jax.ShapeDtypeStruct((B, 1, P), jnp.int32),
                   jax.ShapeDtypeStruct((B, 1, 16), jnp.int32)],
    )(ltri, durT)
    return out.reshape(B, 2, NCH, CH), tot.reshape(B, 16)


_XBLK = 1024
_NXBLK = TBL // _XBLK


def _xpad_body(x_ref, out_ref):
    i = pl.program_id(0)
    out_ref[...] = jnp.where(i < VROWS // _XBLK, x_ref[...], 0.0)


def _build_xpad(x_flat):
    return pl.pallas_call(
        _xpad_body,
        grid=(_NXBLK,),
        in_specs=[pl.BlockSpec(
            (_XBLK, D), lambda i: (jnp.minimum(i, VROWS // _XBLK - 1), 0))],
        out_specs=pl.BlockSpec((_XBLK, D), lambda i: (i, 0)),
        out_shape=jax.ShapeDtypeStruct((TBL, D), jnp.float32),
    )(x_flat)


NBUF = 2


@functools.lru_cache(maxsize=None)
def _make_sc_gather():
    mesh = plsc.VectorSubcoreMesh(
        core_axis_name="c", subcore_axis_name="s",
        num_cores=2, num_subcores=16)

    @functools.partial(
        pl.kernel,
        out_type=jax.ShapeDtypeStruct((NROWS, D), jnp.float32),
        mesh=mesh,
        scratch_types=[
            pltpu.VMEM((NCH, CH), jnp.int32),
            pltpu.VMEM((16,), jnp.int32),
            pltpu.VMEM((NBUF, CH, D), jnp.float32),
            pltpu.VMEM((CH, D), jnp.float32),
            pltpu.SemaphoreType.DMA((NBUF,)),
            pltpu.SemaphoreType.DMA((NBUF,)),
        ],
    )
    def _sc_gather(xpad_hbm, idx_hbm, tot_hbm, out_hbm,
                   idx_v, tot_v, rows_v, zbuf, gsem, wsem):
        wid = lax.axis_index("s") * 2 + lax.axis_index("c")
        b = wid % B
        h = wid // B
        pltpu.sync_copy(idx_hbm.at[b, h], idx_v)
        pltpu.sync_copy(tot_hbm.at[b], tot_v)
        pltpu.sync_copy(xpad_hbm.at[pl.ds(VROWS, CH)], zbuf)
        tot = tot_v[...]

        def g_copy(j):
            return pltpu.make_async_copy(
                xpad_hbm.at[idx_v.at[j]], rows_v.at[j % NBUF],
                gsem.at[j % NBUF])

        def dst(j):
            return out_hbm.at[pl.ds(b * P + (h + 2 * j) * CH, CH)]

        def w_rows(j):
            return pltpu.make_async_copy(
                rows_v.at[j % NBUF], dst(j), wsem.at[j % NBUF])

        def w_zero(j):
            return pltpu.make_async_copy(zbuf, dst(j), wsem.at[j % NBUF])

        t = tot[0]
        conds = [t > (h + 2 * j) * CH for j in range(NCH)]

        @pl.when(conds[0])
        def _():
            g_copy(0).start()

        for j in range(NCH):
            if j + 1 < NCH:
                if j - 1 >= 0:
                    w_rows(j - 1).wait()

                @pl.when(conds[j + 1])
                def _(jj=j + 1):
                    g_copy(jj).start()

            @pl.when(conds[j])
            def _(jj=j):
                g_copy(jj).wait()
                w_rows(jj).start()

            @pl.when(jnp.logical_not(conds[j]))
            def _(jj=j):
                w_zero(jj).start()

        w_rows(NCH - 2).wait()
        w_rows(NCH - 1).wait()

    return _sc_gather


def kernel(x, duration_predictor_output, max_len):
    x_flat = x.reshape(VROWS, D)
    xpad = _build_xpad(x_flat)
    idx3, tot2 = _build_idx(duration_predictor_output)
    out = _make_sc_gather()(xpad, idx3, tot2)
    return out.reshape(B, P, D)


# speedup vs baseline: 67.2332x; 1.0331x over previous
import functools

import jax
import jax.numpy as jnp
from jax import lax
from jax.experimental import pallas as pl
from jax.experimental.pallas import tpu as pltpu
from jax.experimental.pallas import tpu_sc as plsc

B, T, D, P = 16, 512, 256, 2048
NROWS = B * P
VROWS = B * T
PAD = 1024
TBL = VROWS + PAD

NW = 32
ROWS_PER_W = NROWS // NW
CH = 128
NCH = ROWS_PER_W // CH


def _idx_body(ltri_ref, durT_ref, idx_ref, tot_ref):
    durT = durT_ref[...].astype(jnp.float32)
    cum = jnp.dot(ltri_ref[...], durT,
                  preferred_element_type=jnp.float32)
    m = lax.broadcasted_iota(jnp.int32, (1, P), 1)
    pos_i = (m // 1024 + 2 * ((m % 1024) // CH)) * CH + m % CH
    pos_row = pos_i.astype(jnp.float32)
    zrow = VROWS + (pos_i % PAD)
    ones_row = jnp.full((1, T), 1.0, dtype=jnp.float32)
    for b in range(B):
        cum_b = lax.slice(cum, (0, b), (T, b + 1))
        cmp = (cum_b <= pos_row).astype(jnp.float32)
        idx = jnp.dot(ones_row, cmp,
                      preferred_element_type=jnp.float32
                      ).astype(jnp.int32)
        total = lax.slice(cum_b, (T - 1, 0), (T, 1))
        valid = pos_row < total
        flat = jnp.where(valid,
                         b * T + jnp.minimum(idx, T - 1),
                         zrow).astype(jnp.int32)
        idx_ref[b] = flat
        tot_ref[b] = jnp.broadcast_to(total.astype(jnp.int32), (1, 16))


def _build_idx(duration):
    durT = duration.T
    ltri = jnp.tri(T, dtype=jnp.float32)
    out, tot = pl.pallas_call(
        _idx_body,
        out_shape=[jax.ShapeDtypeStruct((B, 1, P), jnp.int32),
                   jax.ShapeDtypeStruct((B, 1, 16), jnp.int32)],
    )(ltri, durT)
    return out.reshape(B, 2, NCH, CH), tot.reshape(B, 16)


_XBLK = 1024
_NXBLK = TBL // _XBLK


def _xpad_body(x_ref, out_ref):
    i = pl.program_id(0)
    out_ref[...] = jnp.where(i < VROWS // _XBLK, x_ref[...], 0.0)


def _build_xpad(x_flat):
    return pl.pallas_call(
        _xpad_body,
        grid=(_NXBLK,),
        in_specs=[pl.BlockSpec(
            (_XBLK, D), lambda i: (jnp.minimum(i, VROWS // _XBLK - 1), 0))],
        out_specs=pl.BlockSpec((_XBLK, D), lambda i: (i, 0)),
        out_shape=jax.ShapeDtypeStruct((TBL, D), jnp.float32),
    )(x_flat)


NBUF = 2


@functools.lru_cache(maxsize=None)
def _make_sc_gather():
    mesh = plsc.VectorSubcoreMesh(
        core_axis_name="c", subcore_axis_name="s",
        num_cores=2, num_subcores=16)

    @functools.partial(
        pl.kernel,
        out_type=jax.ShapeDtypeStruct((NROWS, D), jnp.float32),
        mesh=mesh,
        scratch_types=[
            pltpu.VMEM((NCH, CH), jnp.int32),
            pltpu.VMEM((16,), jnp.int32),
            pltpu.VMEM((NBUF, CH, D), jnp.float32),
            pltpu.VMEM((CH, D), jnp.float32),
            pltpu.SemaphoreType.DMA((NBUF,)),
            pltpu.SemaphoreType.DMA((NBUF,)),
        ],
    )
    def _sc_gather(xpad_hbm, idx_hbm, tot_hbm, out_hbm,
                   idx_v, tot_v, rows_v, zbuf, gsem, wsem):
        wid = lax.axis_index("s") * 2 + lax.axis_index("c")
        b = wid % B
        h = wid // B
        pltpu.sync_copy(idx_hbm.at[b, h], idx_v)
        pltpu.sync_copy(tot_hbm.at[b], tot_v)
        pltpu.sync_copy(xpad_hbm.at[pl.ds(VROWS, CH)], zbuf)
        tot = tot_v[...]

        def g_copy(j):
            return pltpu.make_async_copy(
                xpad_hbm.at[idx_v.at[j]], rows_v.at[j % NBUF],
                gsem.at[j % NBUF])

        def dst(j):
            return out_hbm.at[pl.ds(b * P + (h + 2 * j) * CH, CH)]

        def w_rows(j):
            return pltpu.make_async_copy(
                rows_v.at[j % NBUF], dst(j), wsem.at[j % NBUF])

        def w_zero(j):
            return pltpu.make_async_copy(zbuf, dst(j), wsem.at[j % NBUF])

        t = tot[0]
        conds = [t > (h + 2 * j) * CH for j in range(NCH)]

        @pl.when(conds[0])
        def _():
            g_copy(0).start()

        for j in range(NCH):
            if j + 1 < NCH:
                if j - 1 >= 0:
                    w_rows(j - 1).wait()

                @pl.when(conds[j + 1])
                def _(jj=j + 1):
                    g_copy(jj).start()

            @pl.when(conds[j])
            def _(jj=j):
                g_copy(jj).wait()
                w_rows(jj).start()

            @pl.when(jnp.logical_not(conds[j]))
            def _(jj=j):
                w_zero(jj).start()

        w_rows(NCH - 2).wait()
        w_rows(NCH - 1).wait()

    return _sc_gather


def kernel(x, duration_predictor_output, max_len):
    x_flat = x.reshape(VROWS, D)
    xpad = _build_xpad(x_flat)
    idx3, tot2 = _build_idx(duration_predictor_output)
    out = _make_sc_gather()(xpad, idx3, tot2)
    return out.reshape(B, P, D)


# speedup vs baseline: 69.1841x; 1.0290x over previous
import functools

import jax
import jax.numpy as jnp
from jax import lax
from jax.experimental import pallas as pl
from jax.experimental.pallas import tpu as pltpu
from jax.experimental.pallas import tpu_sc as plsc

B, T, D, P = 16, 512, 256, 2048
NROWS = B * P
VROWS = B * T
PAD = 1024
TBL = VROWS + PAD

NW = 32
ROWS_PER_W = NROWS // NW
CH = 128
NCH = ROWS_PER_W // CH


def _prep_body(ltri_ref, durT_ref, x_ref, idx_ref, tot_ref, xpad_ref):
    i = pl.program_id(0)
    xpad_ref[...] = jnp.where(i < VROWS // _XBLK, x_ref[...], 0.0)

    @pl.when(i == 0)
    def _():
        _idx_compute(ltri_ref, durT_ref, idx_ref, tot_ref)


def _idx_compute(ltri_ref, durT_ref, idx_ref, tot_ref):
    durT = durT_ref[...].astype(jnp.float32)
    cum = jnp.dot(ltri_ref[...], durT,
                  preferred_element_type=jnp.float32)
    m = lax.broadcasted_iota(jnp.int32, (1, P), 1)
    pos_i = (m // 1024 + 2 * ((m % 1024) // CH)) * CH + m % CH
    pos_row = pos_i.astype(jnp.float32)
    zrow = VROWS + (pos_i % PAD)
    ones_row = jnp.full((1, T), 1.0, dtype=jnp.float32)
    for b in range(B):
        cum_b = lax.slice(cum, (0, b), (T, b + 1))
        cmp = (cum_b <= pos_row).astype(jnp.float32)
        idx = jnp.dot(ones_row, cmp,
                      preferred_element_type=jnp.float32
                      ).astype(jnp.int32)
        total = lax.slice(cum_b, (T - 1, 0), (T, 1))
        valid = pos_row < total
        flat = jnp.where(valid,
                         b * T + jnp.minimum(idx, T - 1),
                         zrow).astype(jnp.int32)
        idx_ref[b] = flat
        tot_ref[b] = jnp.broadcast_to(total.astype(jnp.int32), (1, 16))


_XBLK = 1024
_NXBLK = TBL // _XBLK


def _build_prep(duration, x_flat):
    durT = duration.T
    ltri = jnp.tri(T, dtype=jnp.float32)
    idx, tot, xpad = pl.pallas_call(
        _prep_body,
        grid=(_NXBLK,),
        in_specs=[
            pl.BlockSpec((T, T), lambda i: (0, 0)),
            pl.BlockSpec((T, B), lambda i: (0, 0)),
            pl.BlockSpec(
                (_XBLK, D), lambda i: (jnp.minimum(i, VROWS // _XBLK - 1), 0)),
        ],
        out_specs=[
            pl.BlockSpec((B, 1, P), lambda i: (0, 0, 0)),
            pl.BlockSpec((B, 1, 16), lambda i: (0, 0, 0)),
            pl.BlockSpec((_XBLK, D), lambda i: (i, 0)),
        ],
        out_shape=[jax.ShapeDtypeStruct((B, 1, P), jnp.int32),
                   jax.ShapeDtypeStruct((B, 1, 16), jnp.int32),
                   jax.ShapeDtypeStruct((TBL, D), jnp.float32)],
    )(ltri, durT, x_flat)
    return idx.reshape(B, 2, NCH, CH), tot.reshape(B, 16), xpad


NBUF = 2


@functools.lru_cache(maxsize=None)
def _make_sc_gather():
    mesh = plsc.VectorSubcoreMesh(
        core_axis_name="c", subcore_axis_name="s",
        num_cores=2, num_subcores=16)

    @functools.partial(
        pl.kernel,
        out_type=jax.ShapeDtypeStruct((NROWS, D), jnp.float32),
        mesh=mesh,
        scratch_types=[
            pltpu.VMEM((NCH, CH), jnp.int32),
            pltpu.VMEM((16,), jnp.int32),
            pltpu.VMEM((NBUF, CH, D), jnp.float32),
            pltpu.VMEM((CH, D), jnp.float32),
            pltpu.SemaphoreType.DMA((NBUF,)),
            pltpu.SemaphoreType.DMA((NBUF,)),
        ],
    )
    def _sc_gather(xpad_hbm, idx_hbm, tot_hbm, out_hbm,
                   idx_v, tot_v, rows_v, zbuf, gsem, wsem):
        wid = lax.axis_index("s") * 2 + lax.axis_index("c")
        b = wid % B
        h = wid // B
        pltpu.sync_copy(idx_hbm.at[b, h], idx_v)
        pltpu.sync_copy(tot_hbm.at[b], tot_v)
        pltpu.sync_copy(xpad_hbm.at[pl.ds(VROWS, CH)], zbuf)
        tot = tot_v[...]

        def g_copy(j):
            return pltpu.make_async_copy(
                xpad_hbm.at[idx_v.at[j]], rows_v.at[j % NBUF],
                gsem.at[j % NBUF])

        def dst(j):
            return out_hbm.at[pl.ds(b * P + (h + 2 * j) * CH, CH)]

        def w_rows(j):
            return pltpu.make_async_copy(
                rows_v.at[j % NBUF], dst(j), wsem.at[j % NBUF])

        def w_zero(j):
            return pltpu.make_async_copy(zbuf, dst(j), wsem.at[j % NBUF])

        t = tot[0]
        conds = [t > (h + 2 * j) * CH for j in range(NCH)]

        @pl.when(conds[0])
        def _():
            g_copy(0).start()

        for j in range(NCH):
            if j + 1 < NCH:
                if j - 1 >= 0:
                    w_rows(j - 1).wait()

                @pl.when(conds[j + 1])
                def _(jj=j + 1):
                    g_copy(jj).start()

            @pl.when(conds[j])
            def _(jj=j):
                g_copy(jj).wait()
                w_rows(jj).start()

            @pl.when(jnp.logical_not(conds[j]))
            def _(jj=j):
                w_zero(jj).start()

        w_rows(NCH - 2).wait()
        w_rows(NCH - 1).wait()

    return _sc_gather


def kernel(x, duration_predictor_output, max_len):
    x_flat = x.reshape(VROWS, D)
    idx3, tot2, xpad = _build_prep(duration_predictor_output, x_flat)
    out = _make_sc_gather()(xpad, idx3, tot2)
    return out.reshape(B, P, D)
